# baseline (speedup 1.0000x reference)
import functools

import jax
import jax.numpy as jnp
from jax.experimental import pallas as pl
from jax.experimental.pallas import tpu as pltpu

ALPHA = 0.2
PREP_STEPS = 5
ATT2_STEPS = 5


def _prep_attn1_body(x_ref, w1_ref, as1_ref, an1_ref, adj_ref, m_ref,
                     w2_ref, an2_ref, comb_ref, mm_ref,
                     hb_s, s1_s, n1c_s, n1_s):
    i = pl.program_id(0)

    @pl.when(i < PREP_STEPS)
    def _prep():
        rb = x_ref.shape[0]
        h = jnp.dot(x_ref[...], w1_ref[...],
                    preferred_element_type=jnp.float32)
        hb_s[pl.ds(i * rb, rb), :] = h.astype(jnp.bfloat16)
        s1_s[pl.ds(i * rb, rb), :] = jnp.dot(
            h, as1_ref[...], preferred_element_type=jnp.float32)
        n1c_s[pl.ds(i * rb, rb), :] = jnp.dot(
            h, an1_ref[...], preferred_element_type=jnp.float32)

    @pl.when(i == PREP_STEPS)
    def _xpose():
        n1_s[...] = n1c_s[...].T

    @pl.when(i >= PREP_STEPS)
    def _attn():
        j = i - PREP_STEPS
        adj = adj_ref[...]
        m = m_ref[...]
        rb = adj.shape[0]
        mm_ref[...] = jnp.where(adj > 0, m, -1.0).astype(jnp.bfloat16)
        s = s1_s[pl.ds(j * rb, rb), :]
        dense = (s + n1_s[...]) * m
        dense = jnp.maximum(dense, ALPHA * dense)
        p = jnp.exp(dense) * adj
        denom = jnp.sum(p, axis=1, keepdims=True)
        hp = jnp.dot(p.astype(jnp.bfloat16), hb_s[...],
                     preferred_element_type=jnp.float32)
        hp = hp / denom
        hp = jnp.where(hp > 0, hp, jnp.exp(hp) - 1.0)
        h2 = jnp.dot(hp, w2_ref[...], preferred_element_type=jnp.float32)
        comb_ref[:, :16] = h2
        comb_ref[:, 16:17] = jnp.dot(h2, an2_ref[...],
                                     preferred_element_type=jnp.float32)
        comb_ref[:, 17:] = jnp.zeros((rb, 7), jnp.float32)


def _attn2_apred_body(mm_ref, comb_ref, as_ref, nrow_ref, c_ref,
                      z_ref, q_ref, apred_ref, z_s):
    i = pl.program_id(0)

    @pl.when(i < ATT2_STEPS)
    def _attn2():
        mm = mm_ref[...]
        rb = mm.shape[0]
        comb = comb_ref[...]
        hrows = comb_ref[pl.ds(i * rb, rb), :16]
        s = jnp.dot(hrows, as_ref[...], preferred_element_type=jnp.float32)
        b = s.astype(jnp.bfloat16) + nrow_ref[...]
        dense = b * mm
        dense = jnp.maximum(dense, jnp.bfloat16(ALPHA) * dense)
        p = jnp.where(mm >= 0, jnp.exp(dense), jnp.bfloat16(0.0))
        nfull = comb.shape[0]
        h2e = jnp.concatenate(
            [comb[:, :16].astype(jnp.bfloat16),
             jnp.ones((nfull, 8), jnp.bfloat16)], axis=1)
        hpe = jnp.dot(p, h2e, preferred_element_type=jnp.float32)
        hp = hpe[:, :16]
        denom = hpe[:, 16:17]
        hp = hp / denom
        hp = jnp.where(hp > 0, hp, jnp.exp(hp) - 1.0)
        norm = jnp.sqrt(jnp.sum(hp * hp, axis=1, keepdims=True))
        z = hp / jnp.maximum(norm, 1e-12)
        z_ref[...] = z
        z_s[pl.ds(i * rb, rb), :] = z
        c = c_ref[...]
        zn = jnp.sum(z * z, axis=1, keepdims=True)
        cn = jnp.sum(c * c, axis=1, keepdims=True).T
        cross = jax.lax.dot_general(z, c, (((1,), (1,)), ((), ())),
                                    preferred_element_type=jnp.float32)
        dist2 = zn + cn - 2.0 * cross
        qv = 1.0 / (1.0 + dist2)
        q_ref[...] = qv / jnp.sum(qv, axis=1, keepdims=True)

    @pl.when(i >= ATT2_STEPS)
    def _apred():
        j = i - ATT2_STEPS
        rb = apred_ref.shape[0]
        zrows = z_s[pl.ds(j * rb, rb), :]
        g = jax.lax.dot_general(zrows, z_s[...],
                                (((1,), (1,)), ((), ())),
                                preferred_element_type=jnp.float32)
        apred_ref[...] = jax.nn.sigmoid(g)


@functools.partial(jax.jit, static_argnums=())
def kernel(x, adj, M, W1, a_self1, a_neighs1, W2, a_self2, a_neighs2, cluster):
    N, D = x.shape
    H = W1.shape[1]
    E = W2.shape[1]
    K = cluster.shape[0]
    f32 = jnp.float32
    bf16 = jnp.bfloat16

    RBP = N // PREP_STEPS
    RB = 200
    NA1 = N // RB

    comb, Mm = pl.pallas_call(
        _prep_attn1_body,
        grid=(PREP_STEPS + NA1,),
        in_specs=[
            pl.BlockSpec((RBP, D), lambda i: (jnp.minimum(i, PREP_STEPS - 1), 0)),
            pl.BlockSpec((D, H), lambda i: (0, 0)),
            pl.BlockSpec((H, 1), lambda i: (0, 0)),
            pl.BlockSpec((H, 1), lambda i: (0, 0)),
            pl.BlockSpec((RB, N), lambda i: (jnp.maximum(i - PREP_STEPS, 0), 0)),
            pl.BlockSpec((RB, N), lambda i: (jnp.maximum(i - PREP_STEPS, 0), 0)),
            pl.BlockSpec((H, E), lambda i: (0, 0)),
            pl.BlockSpec((E, 1), lambda i: (0, 0)),
        ],
        out_specs=[
            pl.BlockSpec((RB, E + 8), lambda i: (jnp.maximum(i - PREP_STEPS, 0), 0)),
            pl.BlockSpec((RB, N), lambda i: (jnp.maximum(i - PREP_STEPS, 0), 0)),
        ],
        out_shape=[
            jax.ShapeDtypeStruct((N, E + 8), f32),
            jax.ShapeDtypeStruct((N, N), bf16),
        ],
        scratch_shapes=[
            pltpu.VMEM((N, H), bf16),
            pltpu.VMEM((N, 1), f32),
            pltpu.VMEM((N, 1), f32),
            pltpu.VMEM((1, N), f32),
        ],
        compiler_params=pltpu.CompilerParams(
            dimension_semantics=("arbitrary",)),
    )(x, W1, a_self1, a_neighs1, adj, M, W2, a_neighs2)
    n2row = comb[:, 16:17].T.astype(bf16)

    RB2 = N // ATT2_STEPS
    RBA = 200
    NAP = N // RBA

    z, q, a_pred = pl.pallas_call(
        _attn2_apred_body,
        grid=(ATT2_STEPS + NAP,),
        in_specs=[
            pl.BlockSpec((RB2, N), lambda i: (jnp.minimum(i, ATT2_STEPS - 1), 0)),
            pl.BlockSpec((N, E + 8), lambda i: (0, 0)),
            pl.BlockSpec((E, 1), lambda i: (0, 0)),
            pl.BlockSpec((1, N), lambda i: (0, 0)),
            pl.BlockSpec((K, E), lambda i: (0, 0)),
        ],
        out_specs=[
            pl.BlockSpec((RB2, E), lambda i: (jnp.minimum(i, ATT2_STEPS - 1), 0)),
            pl.BlockSpec((RB2, K), lambda i: (jnp.minimum(i, ATT2_STEPS - 1), 0)),
            pl.BlockSpec((RBA, N), lambda i: (jnp.maximum(i - ATT2_STEPS, 0), 0)),
        ],
        out_shape=[
            jax.ShapeDtypeStruct((N, E), f32),
            jax.ShapeDtypeStruct((N, K), f32),
            jax.ShapeDtypeStruct((N, N), f32),
        ],
        scratch_shapes=[
            pltpu.VMEM((N, E), f32),
        ],
        compiler_params=pltpu.CompilerParams(
            dimension_semantics=("arbitrary",)),
    )(Mm, comb, a_self2, n2row, cluster)

    return (a_pred, z, q)


# speedup vs baseline: 1.0018x; 1.0018x over previous
import functools

import jax
import jax.numpy as jnp
from jax.experimental import pallas as pl
from jax.experimental.pallas import tpu as pltpu

ALPHA = 0.2


def _prep1_body(x_ref, w_ref, as_ref, an_ref, hb_ref, sn_ref):
    h = jnp.dot(x_ref[...], w_ref[...], preferred_element_type=jnp.float32)
    hb_ref[...] = h.astype(jnp.bfloat16)
    sn_ref[:, 0:1] = jnp.dot(h, as_ref[...], preferred_element_type=jnp.float32)
    sn_ref[:, 1:2] = jnp.dot(h, an_ref[...], preferred_element_type=jnp.float32)
    sn_ref[:, 2:] = jnp.zeros((h.shape[0], 6), jnp.float32)


def _attn1_body(adj_ref, m_ref, hb_ref, sn_ref, nrow_ref,
                w2_ref, an2_ref, comb_ref, mm_ref):
    adj = adj_ref[...]
    m = m_ref[...]
    mm_ref[...] = jnp.where(adj > 0, m, -1.0).astype(jnp.bfloat16)
    rb = adj.shape[0]
    s = sn_ref[pl.ds(pl.program_id(0) * rb, rb), 0:1]
    dense = (s + nrow_ref[...]) * m
    dense = jnp.maximum(dense, ALPHA * dense)
    p = jnp.exp(dense) * adj
    denom = jnp.sum(p, axis=1, keepdims=True)
    hp = jnp.dot(p.astype(jnp.bfloat16), hb_ref[...],
                 preferred_element_type=jnp.float32)
    hp = hp / denom
    hp = jnp.where(hp > 0, hp, jnp.exp(hp) - 1.0)
    h2 = jnp.dot(hp, w2_ref[...], preferred_element_type=jnp.float32)
    comb_ref[:, :16] = h2
    comb_ref[:, 16:17] = jnp.dot(h2, an2_ref[...],
                                 preferred_element_type=jnp.float32)
    comb_ref[:, 17:] = jnp.zeros((rb, 7), jnp.float32)


def _attn2_body(mm_ref, comb_ref, as_ref, nrow_ref, c_ref, z_ref, q_ref):
    mm = mm_ref[...]
    rb = mm.shape[0]
    comb = comb_ref[...]
    hrows = comb_ref[pl.ds(pl.program_id(0) * rb, rb), :16]
    s = jnp.dot(hrows, as_ref[...], preferred_element_type=jnp.float32)
    b = s.astype(jnp.bfloat16) + nrow_ref[...]
    dense = b * mm
    dense = jnp.maximum(dense, jnp.bfloat16(ALPHA) * dense)
    p = jnp.where(mm >= 0, jnp.exp(dense), jnp.bfloat16(0.0))
    nfull = comb.shape[0]
    h2e = jnp.concatenate(
        [comb[:, :16].astype(jnp.bfloat16),
         jnp.ones((nfull, 8), jnp.bfloat16)], axis=1)
    hpe = jnp.dot(p, h2e, preferred_element_type=jnp.float32)
    hp = hpe[:, :16]
    denom = hpe[:, 16:17]
    hp = hp / denom
    hp = jnp.where(hp > 0, hp, jnp.exp(hp) - 1.0)
    norm = jnp.sqrt(jnp.sum(hp * hp, axis=1, keepdims=True))
    z = hp / jnp.maximum(norm, 1e-12)
    z_ref[...] = z
    c = c_ref[...]
    zn = jnp.sum(z * z, axis=1, keepdims=True)
    cn = jnp.sum(c * c, axis=1, keepdims=True).T
    cross = jax.lax.dot_general(z, c, (((1,), (1,)), ((), ())),
                                preferred_element_type=jnp.float32)
    dist2 = zn + cn - 2.0 * cross
    qv = 1.0 / (1.0 + dist2)
    q_ref[...] = qv / jnp.sum(qv, axis=1, keepdims=True)


def _apred_body(zfull_ref, out_ref):
    rb = out_ref.shape[0]
    zrows = zfull_ref[pl.ds(pl.program_id(0) * rb, rb), :]
    g = jax.lax.dot_general(zrows, zfull_ref[...],
                            (((1,), (1,)), ((), ())),
                            preferred_element_type=jnp.float32)
    out_ref[...] = jax.nn.sigmoid(g)


@functools.partial(jax.jit, static_argnums=())
def kernel(x, adj, M, W1, a_self1, a_neighs1, W2, a_self2, a_neighs2, cluster):
    N, D = x.shape
    H = W1.shape[1]
    E = W2.shape[1]
    K = cluster.shape[0]
    f32 = jnp.float32
    bf16 = jnp.bfloat16

    RBP = 1000
    h1b, sn1 = pl.pallas_call(
        _prep1_body,
        grid=(N // RBP,),
        in_specs=[
            pl.BlockSpec((RBP, D), lambda i: (i, 0)),
            pl.BlockSpec((D, H), lambda i: (0, 0)),
            pl.BlockSpec((H, 1), lambda i: (0, 0)),
            pl.BlockSpec((H, 1), lambda i: (0, 0)),
        ],
        out_specs=[
            pl.BlockSpec((RBP, H), lambda i: (i, 0)),
            pl.BlockSpec((RBP, 8), lambda i: (i, 0)),
        ],
        out_shape=[
            jax.ShapeDtypeStruct((N, H), bf16),
            jax.ShapeDtypeStruct((N, 8), f32),
        ],
        compiler_params=pltpu.CompilerParams(
            dimension_semantics=("parallel",)),
    )(x, W1, a_self1, a_neighs1)
    n1row = sn1[:, 1:2].T

    RB = 200
    comb, Mm = pl.pallas_call(
        _attn1_body,
        grid=(N // RB,),
        in_specs=[
            pl.BlockSpec((RB, N), lambda i: (i, 0)),
            pl.BlockSpec((RB, N), lambda i: (i, 0)),
            pl.BlockSpec((N, H), lambda i: (0, 0)),
            pl.BlockSpec((N, 8), lambda i: (0, 0)),
            pl.BlockSpec((1, N), lambda i: (0, 0)),
            pl.BlockSpec((H, E), lambda i: (0, 0)),
            pl.BlockSpec((E, 1), lambda i: (0, 0)),
        ],
        out_specs=[
            pl.BlockSpec((RB, E + 8), lambda i: (i, 0)),
            pl.BlockSpec((RB, N), lambda i: (i, 0)),
        ],
        out_shape=[
            jax.ShapeDtypeStruct((N, E + 8), f32),
            jax.ShapeDtypeStruct((N, N), bf16),
        ],
        compiler_params=pltpu.CompilerParams(
            dimension_semantics=("parallel",)),
    )(adj, M, h1b, sn1, n1row, W2, a_neighs2)
    n2row = comb[:, 16:17].T.astype(bf16)

    RB2 = 1000
    z, q = pl.pallas_call(
        _attn2_body,
        grid=(N // RB2,),
        in_specs=[
            pl.BlockSpec((RB2, N), lambda i: (i, 0)),
            pl.BlockSpec((N, E + 8), lambda i: (0, 0)),
            pl.BlockSpec((E, 1), lambda i: (0, 0)),
            pl.BlockSpec((1, N), lambda i: (0, 0)),
            pl.BlockSpec((K, E), lambda i: (0, 0)),
        ],
        out_specs=[
            pl.BlockSpec((RB2, E), lambda i: (i, 0)),
            pl.BlockSpec((RB2, K), lambda i: (i, 0)),
        ],
        out_shape=[
            jax.ShapeDtypeStruct((N, E), f32),
            jax.ShapeDtypeStruct((N, K), f32),
        ],
        compiler_params=pltpu.CompilerParams(
            dimension_semantics=("parallel",)),
    )(Mm, comb, a_self2, n2row, cluster)

    RBA = 1000
    a_pred = pl.pallas_call(
        _apred_body,
        grid=(N // RBA,),
        in_specs=[
            pl.BlockSpec((N, E), lambda i: (0, 0)),
        ],
        out_specs=pl.BlockSpec((RBA, N), lambda i: (i, 0)),
        out_shape=jax.ShapeDtypeStruct((N, N), f32),
        compiler_params=pltpu.CompilerParams(
            dimension_semantics=("parallel",)),
    )(z)

    return (a_pred, z, q)
